# Initial kernel scaffold; baseline (speedup 1.0000x reference)
import jax
import jax.numpy as jnp
from jax.experimental import pallas as pl


def kernel(pos, batch, params):
    raise NotImplementedError("write your pallas kernel here")


# speedup vs baseline: 5.4086x; 5.4086x over previous
import functools

import numpy as np
import jax
import jax.numpy as jnp
from jax.experimental import pallas as pl

_F32 = jnp.float32
_I32 = jnp.int32
_BIG_I = 2**30
_INF = 3e38
_HI = jax.lax.Precision.HIGHEST


def _dot(a, b):
    return jax.lax.dot_general(a, b, (((1,), (0,)), ((), ())),
                               precision=_HI, preferred_element_type=_F32)


def _fps_body(n, posr_ref, idx_ref, sel_ref):
    _, S, L = posr_ref.shape
    px = posr_ref[0]
    py = posr_ref[1]
    pz = posr_ref[2]
    iota = (jax.lax.broadcasted_iota(_I32, (S, L), 0) * L
            + jax.lax.broadcasted_iota(_I32, (S, L), 1))

    def pick(c):
        sel = (iota == c).astype(_F32)
        cx = jnp.sum(px * sel, keepdims=True).reshape(1, 1)
        cy = jnp.sum(py * sel, keepdims=True).reshape(1, 1)
        cz = jnp.sum(pz * sel, keepdims=True).reshape(1, 1)
        return cx, cy, cz

    idx_ref[0:1, :] = jnp.zeros((1, 1), _I32)
    cx, cy, cz = pick(jnp.int32(0))
    sel_ref[0:1, :] = jnp.concatenate([cx, cy, cz], axis=1)

    def body(i, st):
        dists, cx, cy, cz = st
        d = (px - cx) ** 2 + (py - cy) ** 2 + (pz - cz) ** 2
        dists = jnp.minimum(dists, d)
        m = jnp.max(dists, keepdims=True).reshape(1, 1)
        cand = jnp.where(dists == m, iota, _BIG_I)
        a = jnp.min(cand, keepdims=True).reshape(1, 1)
        idx_ref[pl.ds(i, 1), :] = a
        cx, cy, cz = pick(a)
        sel_ref[pl.ds(i, 1), :] = jnp.concatenate([cx, cy, cz], axis=1)
        return (dists, cx, cy, cz)

    dists0 = jnp.full((S, L), _INF, _F32)
    jax.lax.fori_loop(1, n, body, (dists0, cx, cy, cz))


def _fps(pos, n):
    N = pos.shape[0]
    posr = pos.T.reshape(3, 8, N // 8)
    idx, sel = pl.pallas_call(
        functools.partial(_fps_body, n),
        out_shape=(jax.ShapeDtypeStruct((n, 1), _I32),
                   jax.ShapeDtypeStruct((n, 3), _F32)),
    )(posr)
    return idx, sel


def _topk_body(r2, k, srcT_ref, dst_ref, nbr_ref, msk_ref):
    Ns = srcT_ref.shape[1]
    B = dst_ref.shape[0]
    sx = srcT_ref[0:1, :]
    sy = srcT_ref[1:2, :]
    sz = srcT_ref[2:3, :]
    dx = dst_ref[:, 0:1]
    dy = dst_ref[:, 1:2]
    dz = dst_ref[:, 2:3]
    d2 = (dx - sx) ** 2 + (dy - sy) ** 2 + (dz - sz) ** 2
    d2 = jnp.where(d2 <= r2, d2, _INF)
    lane = jax.lax.broadcasted_iota(_I32, (B, Ns), 1)
    kl = jax.lax.broadcasted_iota(_I32, (B, k), 1)

    def body(s, st):
        d2, nbrs, msk = st
        m = jnp.min(d2, axis=1, keepdims=True)
        cand = jnp.where(d2 == m, lane, _BIG_I)
        j = jnp.min(cand, axis=1, keepdims=True)
        d2 = jnp.where(cand == j, _INF, d2)
        hit = (kl == s)
        nbrs = jnp.where(hit, j, nbrs)
        msk = jnp.where(hit & (m <= r2), 1.0, msk)
        return (d2, nbrs, msk)

    nbrs0 = jnp.zeros((B, k), _I32)
    msk0 = jnp.zeros((B, k), _F32)
    _, nbrs, msk = jax.lax.fori_loop(0, k, body, (d2, nbrs0, msk0))
    nbr_ref[...] = nbrs
    msk_ref[...] = msk


def _radius_topk(src, dst, r, k=64, block=64):
    Ns = src.shape[0]
    Nd = dst.shape[0]
    srcT = src.T
    grid = Nd // block
    nbr, msk = pl.pallas_call(
        functools.partial(_topk_body, float(np.float32(r * r)), k),
        grid=(grid,),
        in_specs=[
            pl.BlockSpec((3, Ns), lambda i: (0, 0)),
            pl.BlockSpec((block, 3), lambda i: (i, 0)),
        ],
        out_specs=(
            pl.BlockSpec((block, k), lambda i: (i, 0)),
            pl.BlockSpec((block, k), lambda i: (i, 0)),
        ),
        out_shape=(jax.ShapeDtypeStruct((Nd, k), _I32),
                   jax.ShapeDtypeStruct((Nd, k), _F32)),
    )(srcT, dst)
    return nbr, msk


def _conv1_body(mlp_dims, posT2_ref, nbrf_ref, dstrep_ref, msk_ref, *rest):
    w_refs = rest[:-1]
    out_ref = rest[-1]
    M = nbrf_ref.shape[0]
    B, K = msk_ref.shape
    nf = nbrf_ref[...]
    hi = nf // 128
    lo = nf - hi * 128
    oh_hi = (jax.lax.broadcasted_iota(_I32, (M, 64), 1) == hi).astype(_F32)
    inner = _dot(oh_hi, posT2_ref[...])
    oh_lo = (jax.lax.broadcasted_iota(_I32, (M, 128), 1) == lo).astype(_F32)
    gx = jnp.sum(inner[:, 0:128] * oh_lo, axis=1, keepdims=True)
    gy = jnp.sum(inner[:, 128:256] * oh_lo, axis=1, keepdims=True)
    gz = jnp.sum(inner[:, 256:384] * oh_lo, axis=1, keepdims=True)
    relx = gx - dstrep_ref[:, 0:1]
    rely = gy - dstrep_ref[:, 1:2]
    relz = gz - dstrep_ref[:, 2:3]

    W1, b1, W2, b2, W3, b3 = (r[...] for r in w_refs)
    h = jax.nn.relu(relx * W1[0:1, :] + rely * W1[1:2, :] + relz * W1[2:3, :]
                    + b1[0:1, :])
    h = jax.nn.relu(_dot(h, W2) + b2[0:1, :])
    h = jax.nn.relu(_dot(h, W3) + b3[0:1, :])
    D = h.shape[1]
    h = h.reshape(B, K, D)
    h = jnp.where(msk_ref[...][:, :, None] > 0, h, -1e30)
    out = jnp.max(h, axis=1)
    out_ref[...] = jnp.where(out <= -1e29, 0.0, out)


def _conv1(pos, dst, nbr, msk, mlp, block=128):
    Ns = pos.shape[0]
    Nd = dst.shape[0]
    K = nbr.shape[1]
    M = block * K
    posT2 = pos.reshape(64, 128, 3).transpose(0, 2, 1).reshape(64, 384)
    nbrf = nbr.reshape(Nd * K, 1)
    dstrep = jnp.repeat(dst, K, axis=0)
    grid = Nd // block
    dims = [w.shape for w, _ in mlp]
    Dout = mlp[-1][0].shape[1]
    wargs = []
    wspecs = []
    for (W, b) in mlp:
        wargs += [W, b.reshape(1, -1)]
        wspecs += [pl.BlockSpec(W.shape, lambda i: (0, 0)),
                   pl.BlockSpec((1, b.shape[0]), lambda i: (0, 0))]
    out = pl.pallas_call(
        functools.partial(_conv1_body, dims),
        grid=(grid,),
        in_specs=[
            pl.BlockSpec((64, 384), lambda i: (0, 0)),
            pl.BlockSpec((M, 1), lambda i: (i, 0)),
            pl.BlockSpec((M, 3), lambda i: (i, 0)),
            pl.BlockSpec((block, K), lambda i: (i, 0)),
        ] + wspecs,
        out_specs=pl.BlockSpec((block, Dout), lambda i: (i, 0)),
        out_shape=jax.ShapeDtypeStruct((Nd, Dout), _F32),
    )(posT2, nbrf, dstrep, msk, *wargs)
    return out


def _conv_body(Dx, srcfeat_ref, nbrf_ref, dstrep_ref, msk_ref, *rest):
    w_refs = rest[:-1]
    out_ref = rest[-1]
    M = nbrf_ref.shape[0]
    B, K = msk_ref.shape
    Ns = srcfeat_ref.shape[0]
    nf = nbrf_ref[...]
    oh = (jax.lax.broadcasted_iota(_I32, (M, Ns), 1) == nf).astype(_F32)
    g = _dot(oh, srcfeat_ref[...])
    gx = g[:, 0:Dx]
    relx = g[:, Dx:Dx + 1] - dstrep_ref[:, 0:1]
    rely = g[:, Dx + 1:Dx + 2] - dstrep_ref[:, 1:2]
    relz = g[:, Dx + 2:Dx + 3] - dstrep_ref[:, 2:3]

    W1, b1, W2, b2, W3, b3 = (r[...] for r in w_refs)
    h = jax.nn.relu(_dot(gx, W1[0:Dx, :])
                    + relx * W1[Dx:Dx + 1, :] + rely * W1[Dx + 1:Dx + 2, :]
                    + relz * W1[Dx + 2:Dx + 3, :] + b1[0:1, :])
    h = jax.nn.relu(_dot(h, W2) + b2[0:1, :])
    h = jax.nn.relu(_dot(h, W3) + b3[0:1, :])
    D = h.shape[1]
    h = h.reshape(B, K, D)
    h = jnp.where(msk_ref[...][:, :, None] > 0, h, -1e30)
    out = jnp.max(h, axis=1)
    out_ref[...] = jnp.where(out <= -1e29, 0.0, out)


def _conv(x_src, pos_src, dst, nbr, msk, mlp, block):
    Ns = pos_src.shape[0]
    Nd = dst.shape[0]
    K = nbr.shape[1]
    M = block * K
    Dx = x_src.shape[1]
    srcfeat = jnp.concatenate([x_src, pos_src], axis=1)
    nbrf = nbr.reshape(Nd * K, 1)
    dstrep = jnp.repeat(dst, K, axis=0)
    grid = Nd // block
    Dout = mlp[-1][0].shape[1]
    wargs = []
    wspecs = []
    for (W, b) in mlp:
        wargs += [W, b.reshape(1, -1)]
        wspecs += [pl.BlockSpec(W.shape, lambda i: (0, 0)),
                   pl.BlockSpec((1, b.shape[0]), lambda i: (0, 0))]
    out = pl.pallas_call(
        functools.partial(_conv_body, Dx),
        grid=(grid,),
        in_specs=[
            pl.BlockSpec(srcfeat.shape, lambda i: (0, 0)),
            pl.BlockSpec((M, 1), lambda i: (i, 0)),
            pl.BlockSpec((M, 3), lambda i: (i, 0)),
            pl.BlockSpec((block, K), lambda i: (i, 0)),
        ] + wspecs,
        out_specs=pl.BlockSpec((block, Dout), lambda i: (i, 0)),
        out_shape=jax.ShapeDtypeStruct((Nd, Dout), _F32),
    )(srcfeat, nbrf, dstrep, msk, *wargs)
    return out


def _up4(x, n_out):
    n, d = x.shape
    return jnp.broadcast_to(x[:, None, :], (n, 4, d)).reshape(n * 4, d)


def _fp_body(x3_ref, x2_ref, x1_ref, pos_ref, *rest):
    w_refs = rest[:-1]
    out_ref = rest[-1]
    ws = [r[...] for r in w_refs]
    (f3a, f3b1, f3w2, f3b2, f2a, f2b1, f2w2, f2b2,
     f1a, f1b1, f1w2, f1b2, hw1, hb1, hw2, hb2) = ws

    x3 = x3_ref[...]
    x2 = x2_ref[...]
    x1 = x1_ref[...]
    pos = pos_ref[...]

    x2u = _up4(x3, 512)
    h = jax.nn.relu(_dot(x2u, f3a[0:1024, :]) + _dot(x2, f3a[1024:1280, :])
                    + f3b1[0:1, :])
    x2fp = jax.nn.relu(_dot(h, f3w2) + f3b2[0:1, :])

    x1u = _up4(x2fp, 2048)
    h = jax.nn.relu(_dot(x1u, f2a[0:256, :]) + _dot(x1, f2a[256:384, :])
                    + f2b1[0:1, :])
    x1fp = jax.nn.relu(_dot(h, f2w2) + f2b2[0:1, :])

    x0u = _up4(x1fp, 8192)
    h = jax.nn.relu(_dot(x0u, f1a[0:128, :])
                    + pos[:, 0:1] * f1a[128:129, :]
                    + pos[:, 1:2] * f1a[129:130, :]
                    + pos[:, 2:3] * f1a[130:131, :] + f1b1[0:1, :])
    x0fp = jax.nn.relu(_dot(h, f1w2) + f1b2[0:1, :])

    h = jax.nn.relu(_dot(x0fp, hw1) + hb1[0:1, :])
    alpha = _dot(h, hw2) + hb2[0:1, :]
    out_ref[...] = jnp.maximum(alpha, 0.0) + jnp.log1p(jnp.exp(-jnp.abs(alpha)))


def _fp_head(x3, x2, x1, pos, params):
    wargs = []
    for name in ('fp3', 'fp2', 'fp1', 'head'):
        for (W, b) in params[name]:
            wargs += [W, b.reshape(1, -1)]
    out = pl.pallas_call(
        _fp_body,
        out_shape=jax.ShapeDtypeStruct((8192, 1), _F32),
    )(x3, x2, x1, pos, *wargs)
    return out


def kernel(pos, batch, params):
    n = pos.shape[0]
    _, pos1 = _fps(pos, n // 4)
    nbr1, m1 = _radius_topk(pos, pos1, 0.2)
    x1 = _conv1(pos, pos1, nbr1, m1, params['sa1'])

    _, pos2 = _fps(pos1, n // 16)
    nbr2, m2 = _radius_topk(pos1, pos2, 0.4)
    x2 = _conv(x1, pos1, pos2, nbr2, m2, params['sa2'], block=32)

    _, pos3 = _fps(pos2, n // 64)
    nbr3, m3 = _radius_topk(pos2, pos3, 0.8)
    x3 = _conv(x2, pos2, pos3, nbr3, m3, params['sa3'], block=16)

    alpha = _fp_head(x3, x2, x1, pos, params)
    mean = alpha.reshape(1, n, 1).transpose(0, 2, 1)
    std = jnp.ones_like(mean) * 0.01
    return (mean, std)


# speedup vs baseline: 6.1518x; 1.1374x over previous
import functools

import numpy as np
import jax
import jax.numpy as jnp
from jax.experimental import pallas as pl

_F32 = jnp.float32
_I32 = jnp.int32
_BIG_I = 2**30
_INF = 3e38
_BF16 = jnp.bfloat16


def _d(a, b):
    return jax.lax.dot_general(a, b, (((1,), (0,)), ((), ())),
                               preferred_element_type=_F32)


def _dot(a, b):
    ah = a.astype(_BF16)
    al = (a - ah.astype(_F32)).astype(_BF16)
    bh = b.astype(_BF16)
    bl = (b - bh.astype(_F32)).astype(_BF16)
    return _d(ah, bh) + (_d(al, bh) + _d(ah, bl))


def _dotg(oh, b):
    b0 = b.astype(_BF16)
    r = b - b0.astype(_F32)
    b1 = r.astype(_BF16)
    b2 = (r - b1.astype(_F32)).astype(_BF16)
    return _d(oh, b0) + (_d(oh, b1) + _d(oh, b2))


def _fps_body(n, posr_ref, posrow_ref, idx_ref, sel_ref):
    _, S, L = posr_ref.shape
    px = posr_ref[0]
    py = posr_ref[1]
    pz = posr_ref[2]
    iota = (jax.lax.broadcasted_iota(_I32, (S, L), 0) * L
            + jax.lax.broadcasted_iota(_I32, (S, L), 1))

    idx_ref[0:1, :] = jnp.zeros((1, 1), _I32)
    p0 = posrow_ref[0:1, :]
    sel_ref[0:1, :] = p0

    def body(i, st):
        dists, cx, cy, cz = st
        d = (px - cx) ** 2 + (py - cy) ** 2 + (pz - cz) ** 2
        dists = jnp.minimum(dists, d)
        m = jnp.max(dists, keepdims=True).reshape(1, 1)
        cand = jnp.where(dists == m, iota, _BIG_I)
        a = jnp.min(cand, keepdims=True).reshape(1, 1)
        idx_ref[pl.ds(i, 1), :] = a
        p = posrow_ref[pl.ds(a[0, 0], 1), :]
        sel_ref[pl.ds(i, 1), :] = p
        return (dists, p[0:1, 0:1], p[0:1, 1:2], p[0:1, 2:3])

    dists0 = jnp.full((S, L), _INF, _F32)
    jax.lax.fori_loop(1, n, body,
                      (dists0, p0[0:1, 0:1], p0[0:1, 1:2], p0[0:1, 2:3]))


def _fps(pos, n):
    N = pos.shape[0]
    posr = pos.T.reshape(3, 8, N // 8)
    idx, sel = pl.pallas_call(
        functools.partial(_fps_body, n),
        out_shape=(jax.ShapeDtypeStruct((n, 1), _I32),
                   jax.ShapeDtypeStruct((n, 3), _F32)),
    )(posr, pos)
    return idx, sel


def _topk_body(r2, k, srcT_ref, dst_ref, nbr_ref, msk_ref):
    Ns = srcT_ref.shape[1]
    B = dst_ref.shape[0]
    sx = srcT_ref[0:1, :]
    sy = srcT_ref[1:2, :]
    sz = srcT_ref[2:3, :]
    dx = dst_ref[:, 0:1]
    dy = dst_ref[:, 1:2]
    dz = dst_ref[:, 2:3]
    d2 = (dx - sx) ** 2 + (dy - sy) ** 2 + (dz - sz) ** 2
    d2 = jnp.where(d2 <= r2, d2, _INF)
    lane = jax.lax.broadcasted_iota(_I32, (B, Ns), 1)
    kl = jax.lax.broadcasted_iota(_I32, (B, k), 1)

    def body(s, st):
        d2, nbrs, msk = st
        m = jnp.min(d2, axis=1, keepdims=True)
        cand = jnp.where(d2 == m, lane, _BIG_I)
        j = jnp.min(cand, axis=1, keepdims=True)
        d2 = jnp.where(cand == j, _INF, d2)
        hit = (kl == s)
        nbrs = jnp.where(hit, j, nbrs)
        msk = jnp.where(hit & (m <= r2), 1.0, msk)
        return (d2, nbrs, msk)

    nbrs0 = jnp.zeros((B, k), _I32)
    msk0 = jnp.zeros((B, k), _F32)
    _, nbrs, msk = jax.lax.fori_loop(0, k, body, (d2, nbrs0, msk0))
    nbr_ref[...] = nbrs
    msk_ref[...] = msk


def _radius_topk(src, dst, r, k=64, block=64):
    Ns = src.shape[0]
    Nd = dst.shape[0]
    srcT = src.T
    grid = Nd // block
    nbr, msk = pl.pallas_call(
        functools.partial(_topk_body, float(np.float32(r * r)), k),
        grid=(grid,),
        in_specs=[
            pl.BlockSpec((3, Ns), lambda i: (0, 0)),
            pl.BlockSpec((block, 3), lambda i: (i, 0)),
        ],
        out_specs=(
            pl.BlockSpec((block, k), lambda i: (i, 0)),
            pl.BlockSpec((block, k), lambda i: (i, 0)),
        ),
        out_shape=(jax.ShapeDtypeStruct((Nd, k), _I32),
                   jax.ShapeDtypeStruct((Nd, k), _F32)),
    )(srcT, dst)
    return nbr, msk


def _conv1_body(mlp_dims, posT2_ref, nbrf_ref, dstrep_ref, msk_ref, *rest):
    w_refs = rest[:-1]
    out_ref = rest[-1]
    M = nbrf_ref.shape[0]
    B, K = msk_ref.shape
    nf = nbrf_ref[...]
    hi = nf // 128
    lo = nf - hi * 128
    oh_hi = (jax.lax.broadcasted_iota(_I32, (M, 64), 1) == hi).astype(_BF16)
    inner = _dotg(oh_hi, posT2_ref[...])
    oh_lo = (jax.lax.broadcasted_iota(_I32, (M, 128), 1) == lo).astype(_F32)
    gx = jnp.sum(inner[:, 0:128] * oh_lo, axis=1, keepdims=True)
    gy = jnp.sum(inner[:, 128:256] * oh_lo, axis=1, keepdims=True)
    gz = jnp.sum(inner[:, 256:384] * oh_lo, axis=1, keepdims=True)
    relx = gx - dstrep_ref[:, 0:1]
    rely = gy - dstrep_ref[:, 1:2]
    relz = gz - dstrep_ref[:, 2:3]

    W1, b1, W2, b2, W3, b3 = (r[...] for r in w_refs)
    h = jax.nn.relu(relx * W1[0:1, :] + rely * W1[1:2, :] + relz * W1[2:3, :]
                    + b1[0:1, :])
    h = jax.nn.relu(_dot(h, W2) + b2[0:1, :])
    h = jax.nn.relu(_dot(h, W3) + b3[0:1, :])
    D = h.shape[1]
    h = h.reshape(B, K, D)
    h = jnp.where(msk_ref[...][:, :, None] > 0, h, -1e30)
    out = jnp.max(h, axis=1)
    out_ref[...] = jnp.where(out <= -1e29, 0.0, out)


def _conv1(pos, dst, nbr, msk, mlp, block=128):
    Ns = pos.shape[0]
    Nd = dst.shape[0]
    K = nbr.shape[1]
    M = block * K
    posT2 = pos.reshape(64, 128, 3).transpose(0, 2, 1).reshape(64, 384)
    nbrf = nbr.reshape(Nd * K, 1)
    dstrep = jnp.repeat(dst, K, axis=0)
    grid = Nd // block
    dims = [w.shape for w, _ in mlp]
    Dout = mlp[-1][0].shape[1]
    wargs = []
    wspecs = []
    for (W, b) in mlp:
        wargs += [W, b.reshape(1, -1)]
        wspecs += [pl.BlockSpec(W.shape, lambda i: (0, 0)),
                   pl.BlockSpec((1, b.shape[0]), lambda i: (0, 0))]
    out = pl.pallas_call(
        functools.partial(_conv1_body, dims),
        grid=(grid,),
        in_specs=[
            pl.BlockSpec((64, 384), lambda i: (0, 0)),
            pl.BlockSpec((M, 1), lambda i: (i, 0)),
            pl.BlockSpec((M, 3), lambda i: (i, 0)),
            pl.BlockSpec((block, K), lambda i: (i, 0)),
        ] + wspecs,
        out_specs=pl.BlockSpec((block, Dout), lambda i: (i, 0)),
        out_shape=jax.ShapeDtypeStruct((Nd, Dout), _F32),
    )(posT2, nbrf, dstrep, msk, *wargs)
    return out


def _conv_body(Dx, srcfeat_ref, nbrf_ref, dstrep_ref, msk_ref, *rest):
    w_refs = rest[:-1]
    out_ref = rest[-1]
    M = nbrf_ref.shape[0]
    B, K = msk_ref.shape
    Ns = srcfeat_ref.shape[0]
    nf = nbrf_ref[...]
    oh = (jax.lax.broadcasted_iota(_I32, (M, Ns), 1) == nf).astype(_BF16)
    g = _dotg(oh, srcfeat_ref[...])
    gx = g[:, 0:Dx]
    relx = g[:, Dx:Dx + 1] - dstrep_ref[:, 0:1]
    rely = g[:, Dx + 1:Dx + 2] - dstrep_ref[:, 1:2]
    relz = g[:, Dx + 2:Dx + 3] - dstrep_ref[:, 2:3]

    W1, b1, W2, b2, W3, b3 = (r[...] for r in w_refs)
    h = jax.nn.relu(_dot(gx, W1[0:Dx, :])
                    + relx * W1[Dx:Dx + 1, :] + rely * W1[Dx + 1:Dx + 2, :]
                    + relz * W1[Dx + 2:Dx + 3, :] + b1[0:1, :])
    h = jax.nn.relu(_dot(h, W2) + b2[0:1, :])
    h = jax.nn.relu(_dot(h, W3) + b3[0:1, :])
    D = h.shape[1]
    h = h.reshape(B, K, D)
    h = jnp.where(msk_ref[...][:, :, None] > 0, h, -1e30)
    out = jnp.max(h, axis=1)
    out_ref[...] = jnp.where(out <= -1e29, 0.0, out)


def _conv(x_src, pos_src, dst, nbr, msk, mlp, block):
    Ns = pos_src.shape[0]
    Nd = dst.shape[0]
    K = nbr.shape[1]
    M = block * K
    Dx = x_src.shape[1]
    srcfeat = jnp.concatenate([x_src, pos_src], axis=1)
    nbrf = nbr.reshape(Nd * K, 1)
    dstrep = jnp.repeat(dst, K, axis=0)
    grid = Nd // block
    Dout = mlp[-1][0].shape[1]
    wargs = []
    wspecs = []
    for (W, b) in mlp:
        wargs += [W, b.reshape(1, -1)]
        wspecs += [pl.BlockSpec(W.shape, lambda i: (0, 0)),
                   pl.BlockSpec((1, b.shape[0]), lambda i: (0, 0))]
    out = pl.pallas_call(
        functools.partial(_conv_body, Dx),
        grid=(grid,),
        in_specs=[
            pl.BlockSpec(srcfeat.shape, lambda i: (0, 0)),
            pl.BlockSpec((M, 1), lambda i: (i, 0)),
            pl.BlockSpec((M, 3), lambda i: (i, 0)),
            pl.BlockSpec((block, K), lambda i: (i, 0)),
        ] + wspecs,
        out_specs=pl.BlockSpec((block, Dout), lambda i: (i, 0)),
        out_shape=jax.ShapeDtypeStruct((Nd, Dout), _F32),
    )(srcfeat, nbrf, dstrep, msk, *wargs)
    return out


def _up4(x, n_out):
    n, d = x.shape
    return jnp.broadcast_to(x[:, None, :], (n, 4, d)).reshape(n * 4, d)


def _fp_body(x3_ref, x2_ref, x1_ref, pos_ref, *rest):
    w_refs = rest[:-1]
    out_ref = rest[-1]
    ws = [r[...] for r in w_refs]
    (f3a, f3b1, f3w2, f3b2, f2a, f2b1, f2w2, f2b2,
     f1a, f1b1, f1w2, f1b2, hw1, hb1, hw2, hb2) = ws

    x3 = x3_ref[...]
    x2 = x2_ref[...]
    x1 = x1_ref[...]
    pos = pos_ref[...]

    x2u = _up4(x3, 512)
    h = jax.nn.relu(_dot(x2u, f3a[0:1024, :]) + _dot(x2, f3a[1024:1280, :])
                    + f3b1[0:1, :])
    x2fp = jax.nn.relu(_dot(h, f3w2) + f3b2[0:1, :])

    x1u = _up4(x2fp, 2048)
    h = jax.nn.relu(_dot(x1u, f2a[0:256, :]) + _dot(x1, f2a[256:384, :])
                    + f2b1[0:1, :])
    x1fp = jax.nn.relu(_dot(h, f2w2) + f2b2[0:1, :])

    x0u = _up4(x1fp, 8192)
    h = jax.nn.relu(_dot(x0u, f1a[0:128, :])
                    + pos[:, 0:1] * f1a[128:129, :]
                    + pos[:, 1:2] * f1a[129:130, :]
                    + pos[:, 2:3] * f1a[130:131, :] + f1b1[0:1, :])
    x0fp = jax.nn.relu(_dot(h, f1w2) + f1b2[0:1, :])

    h = jax.nn.relu(_dot(x0fp, hw1) + hb1[0:1, :])
    alpha = _dot(h, hw2) + hb2[0:1, :]
    out_ref[...] = jnp.maximum(alpha, 0.0) + jnp.log1p(jnp.exp(-jnp.abs(alpha)))


def _fp_head(x3, x2, x1, pos, params):
    wargs = []
    for name in ('fp3', 'fp2', 'fp1', 'head'):
        for (W, b) in params[name]:
            wargs += [W, b.reshape(1, -1)]
    out = pl.pallas_call(
        _fp_body,
        out_shape=jax.ShapeDtypeStruct((8192, 1), _F32),
    )(x3, x2, x1, pos, *wargs)
    return out


def kernel(pos, batch, params):
    n = pos.shape[0]
    _, pos1 = _fps(pos, n // 4)
    nbr1, m1 = _radius_topk(pos, pos1, 0.2)
    x1 = _conv1(pos, pos1, nbr1, m1, params['sa1'])

    _, pos2 = _fps(pos1, n // 16)
    nbr2, m2 = _radius_topk(pos1, pos2, 0.4)
    x2 = _conv(x1, pos1, pos2, nbr2, m2, params['sa2'], block=32)

    _, pos3 = _fps(pos2, n // 64)
    nbr3, m3 = _radius_topk(pos2, pos3, 0.8)
    x3 = _conv(x2, pos2, pos3, nbr3, m3, params['sa3'], block=16)

    alpha = _fp_head(x3, x2, x1, pos, params)
    mean = alpha.reshape(1, n, 1).transpose(0, 2, 1)
    std = jnp.ones_like(mean) * 0.01
    return (mean, std)
